# baseline (speedup 1.0000x reference)
import jax
import jax.numpy as jnp
from jax import lax
from jax.experimental import pallas as pl
from jax.experimental.pallas import tpu as pltpu
from jax.experimental.pallas import tpu_sc as plsc

_B, _N, _Q = 2, 4096, 4096
_HID, _K, _POS, _OUT, _CB, _H, _CD = 256, 8, 128, 128, 128, 4, 64
_BN = _B * _N
_BQ = _B * _Q
_TN = 1024
_TK = 256
_TD = 512
_GW = 128
_NC = 1
_QC = _Q // _NC


def _enc_body(xf, we1, be1, we2, be2, wxp, cb, cbt, table, orth_v):
    step = pl.program_id(0)

    @pl.when(step == 0)
    def _orth():
        o_acc = jnp.zeros((1, _CB), jnp.float32)
        for h in range(_H):
            ch = cb[h]
            cht = cbt[h]
            nrm = jnp.sqrt(jnp.sum(ch * ch, axis=1, keepdims=True))
            nrm_t = jnp.sqrt(jnp.sum(cht * cht, axis=0, keepdims=True))
            nc = ch / (nrm + 1e-8)
            nct = cht / (nrm_t + 1e-8)
            cosm = jnp.dot(nc, nct, preferred_element_type=jnp.float32)
            o_acc = o_acc + jnp.sum(cosm * cosm, axis=0, keepdims=True)
        orth_v[...] = o_acc

    h = jnp.maximum(jnp.dot(xf[...], we1[...], preferred_element_type=jnp.float32)
                    + be1[...], 0.0)
    pts = jnp.dot(h, we2[...], preferred_element_type=jnp.float32) + be2[...]
    p = jnp.dot(xf[...], wxp[...], preferred_element_type=jnp.float32)
    table[:, 0:_OUT] = pts
    table[:, _OUT:_OUT + _POS] = p


def _encoder(xf, we1, be1, we2, be2, wxp, cb, cbt):
    grid = (_BN // _TN,)
    return pl.pallas_call(
        _enc_body,
        grid=grid,
        in_specs=[
            pl.BlockSpec((_TN, 3), lambda i: (i, 0)),
            pl.BlockSpec((3, _OUT), lambda i: (0, 0)),
            pl.BlockSpec((1, _OUT), lambda i: (0, 0)),
            pl.BlockSpec((_OUT, _OUT), lambda i: (0, 0)),
            pl.BlockSpec((1, _OUT), lambda i: (0, 0)),
            pl.BlockSpec((3, _POS), lambda i: (0, 0)),
            pl.BlockSpec((_H, _CB, _CD), lambda i: (0, 0, 0)),
            pl.BlockSpec((_H, _CD, _CB), lambda i: (0, 0, 0)),
        ],
        out_specs=[
            pl.BlockSpec((_TN, _OUT + _POS), lambda i: (i, 0)),
            pl.BlockSpec((1, _CB), lambda i: (0, 0)),
        ],
        out_shape=[
            jax.ShapeDtypeStruct((_BN, _OUT + _POS), jnp.float32),
            jax.ShapeDtypeStruct((1, _CB), jnp.float32),
        ],
    )(xf, we1, be1, we2, be2, wxp, cb, cbt)


_NG = 32
_GL = _N // _NG
_T = 3


def _knn_body(q3, xt, idx):
    b = pl.program_id(0)
    qt = q3[0]
    x = xt[0]
    qx = jnp.dot(qt, x, preferred_element_type=jnp.float32)
    qsq = jnp.sum(qt * qt, axis=1, keepdims=True)
    xsq = jnp.sum(x * x, axis=0, keepdims=True)
    d = qsq - 2.0 * qx + xsq
    cols = [d[:, c * _GL:(c + 1) * _GL] for c in range(_NG)]
    liota = lax.broadcasted_iota(jnp.int32, (_TK, _GL), 1)
    cvs, cis = [], []
    for t in range(_T):
        m = cols[0]
        for c in range(1, _NG):
            m = jnp.minimum(m, cols[c])
        ci = jnp.full((_TK, _GL), _NG, jnp.int32)
        for c in range(_NG):
            eq = cols[c] == m
            ci = jnp.minimum(ci, jnp.where(eq, c, _NG))
            cols[c] = jnp.where(eq, jnp.inf, cols[c])
        cvs.append(m)
        cis.append(ci * _GL + liota)
    off = b * _N
    for k in range(_K):
        m2 = cvs[0]
        for t in range(1, _T):
            m2 = jnp.minimum(m2, cvs[t])
        m = jnp.min(m2, axis=1, keepdims=True)
        ik = jnp.full((_TK, 1), _N, jnp.int32)
        for t in range(_T):
            eq = cvs[t] == m
            cand = jnp.min(jnp.where(eq, cis[t], _N), axis=1, keepdims=True)
            ik = jnp.minimum(ik, cand)
            cvs[t] = jnp.where(eq, jnp.inf, cvs[t])
        idx[0, :, k:k + 1] = ik + off


def _knn(q, xt):
    nq = q.shape[1]
    grid = (_B, nq // _TK)
    return pl.pallas_call(
        _knn_body,
        grid=grid,
        in_specs=[
            pl.BlockSpec((1, _TK, 3), lambda b, i: (b, i, 0)),
            pl.BlockSpec((1, 3, _N), lambda b, i: (b, 0, 0)),
        ],
        out_specs=pl.BlockSpec((1, _TK, _K), lambda b, i: (b, i, 0)),
        out_shape=jax.ShapeDtypeStruct((_B, nq, _K), jnp.int32),
    )(q, xt)


def _sc_gather(table, idxt):
    nidx = idxt.shape[1]
    mesh = plsc.VectorSubcoreMesh(core_axis_name="core",
                                  subcore_axis_name="subcore")

    @pl.kernel(out_type=jax.ShapeDtypeStruct((nidx, _OUT + _POS), jnp.float32),
               mesh=mesh)
    def gk(tab_hbm, i_hbm, o_hbm):
        def body(i_vmem, o_vmem):
            pltpu.sync_copy(tab_hbm.at[i_vmem.at[0]], o_vmem)

        pltpu.emit_pipeline(
            body,
            grid=(nidx // _GW,),
            in_specs=[pl.BlockSpec((1, _GW), index_map=lambda i: (0, i))],
            out_specs=[pl.BlockSpec((_GW, _OUT + _POS),
                                    index_map=lambda i: (i, 0))],
            core_axis_name=("core", "subcore"),
            dimension_semantics=(pltpu.PARALLEL,),
        )(i_hbm, o_hbm)

    return gk(table, idxt)


def _dense_body(g, qf, wqd, bpos, w0t, b0, w1t, b1, w2t, b2,
                cb, cbt, csq, w3t, b3, w4t, b4, wot, bo,
                out, commit_v, counts_v):
    step = pl.program_id(0)

    @pl.when(step == 0)
    def _init():
        commit_v[...] = jnp.zeros_like(commit_v)
        counts_v[...] = jnp.zeros_like(counts_v)

    qt = qf[...]
    qpart = jnp.dot(qt, wqd[...], preferred_element_type=jnp.float32) + bpos[...]
    acc = jnp.zeros((_TD, _HID * 2), jnp.float32)
    for k in range(_K):
        gk = g[k]
        posk = jnp.maximum(qpart + gk[:, _OUT:_OUT + _POS],
                           0.0).astype(jnp.bfloat16)
        acc = acc + jnp.dot(posk, w0t[k, 0:_POS, :],
                            preferred_element_type=jnp.float32)
        acc = acc + jnp.dot(gk[:, 0:_OUT].astype(jnp.bfloat16),
                            w0t[k, _POS:_POS + _OUT, :],
                            preferred_element_type=jnp.float32)
    h0 = jnp.maximum(acc + b0[...], 0.0)
    h1 = jnp.maximum(jnp.dot(h0, w1t[...], preferred_element_type=jnp.float32)
                     + b1[...], 0.0)
    f = jnp.maximum(jnp.dot(h1, w2t[...], preferred_element_type=jnp.float32)
                    + b2[...], 0.0)

    iota = lax.broadcasted_iota(jnp.int32, (_TD, _CB), 1)
    quants = []
    for h in range(_H):
        xh = f[:, h * _CD:(h + 1) * _CD]
        xsq = jnp.sum(xh * xh, axis=1, keepdims=True)
        dh = (xsq - 2.0 * jnp.dot(xh, cbt[h], preferred_element_type=jnp.float32)
              + csq[h])
        m = jnp.min(dh, axis=1, keepdims=True)
        sel = jnp.where(dh == m, iota, _CB)
        ind = jnp.min(sel, axis=1, keepdims=True)
        onehot = (iota == ind).astype(jnp.float32)
        quants.append(jnp.dot(onehot, cb[h], preferred_element_type=jnp.float32))
        if h == 0:
            counts_v[...] += jnp.sum(onehot, axis=0, keepdims=True)
    quant = jnp.concatenate(quants, axis=1)
    err2 = (quant - f) ** 2
    commit_v[...] += jnp.sum(err2, axis=0, keepdims=True)

    net = jnp.maximum(jnp.dot(f, w3t[0:_HID, :], preferred_element_type=jnp.float32)
                      + jnp.dot(quant, w3t[_HID:2 * _HID, :],
                                preferred_element_type=jnp.float32)
                      + b3[...], 0.0)
    net = jnp.maximum(jnp.dot(net, w4t[...], preferred_element_type=jnp.float32)
                      + b4[...], 0.0)
    out[...] = jnp.dot(net, wot[...], preferred_element_type=jnp.float32) + bo[...]


def _dense(g3, qf, wqd, bpos, w0t, b0, w1t, b1, w2t, b2,
           cb, cbt, csq, w3t, b3, w4t, b4, wot, bo):
    nr = qf.shape[0]
    grid = (nr // _TD,)
    const = lambda *shape: pl.BlockSpec(shape, lambda i: tuple(0 for _ in shape))
    return pl.pallas_call(
        _dense_body,
        grid=grid,
        in_specs=[
            pl.BlockSpec((_K, _TD, _OUT + _POS), lambda i: (0, i, 0)),
            pl.BlockSpec((_TD, 3), lambda i: (i, 0)),
            const(3, _POS),
            const(1, _POS),
            const(_K, _OUT + _POS, _HID * 2),
            const(1, _HID * 2),
            const(_HID * 2, _HID),
            const(1, _HID),
            const(_HID, _HID),
            const(1, _HID),
            const(_H, _CB, _CD),
            const(_H, _CD, _CB),
            const(_H, 1, _CB),
            const(_HID * 2, _HID),
            const(1, _HID),
            const(_HID, _HID),
            const(1, _HID),
            const(_HID, 3),
            const(1, 3),
        ],
        out_specs=[
            pl.BlockSpec((_TD, 3), lambda i: (i, 0)),
            pl.BlockSpec((1, _HID), lambda i: (0, 0)),
            pl.BlockSpec((1, _CB), lambda i: (0, 0)),
        ],
        out_shape=[
            jax.ShapeDtypeStruct((nr, 3), jnp.float32),
            jax.ShapeDtypeStruct((1, _HID), jnp.float32),
            jax.ShapeDtypeStruct((1, _CB), jnp.float32),
        ],
    )(g3, qf, wqd, bpos, w0t, b0, w1t, b1, w2t, b2,
      cb, cbt, csq, w3t, b3, w4t, b4, wot, bo)


def kernel(q, x, W_enc1, b_enc1, W_enc2, b_enc2, W_pos, b_pos, W0, b0,
           W1, b1, W2, b2, W3, b3, W4, b4, W_out, b_out, codebook):
    f32 = jnp.float32
    bf16 = jnp.bfloat16
    wxp = W_pos[3:6] + W_pos[6:9]
    wqd = W_pos[0:3] - W_pos[6:9]
    w0t = W0.T.reshape(_K, _OUT + _POS, _HID * 2).astype(bf16)
    cbt = jnp.transpose(codebook, (0, 2, 1))
    csq = jnp.sum(codebook * codebook, axis=-1)[:, None, :]

    xf = x.reshape(_BN, 3)
    table, orth_v = _encoder(xf, W_enc1, b_enc1[None, :], W_enc2, b_enc2[None, :],
                             wxp, codebook, cbt)

    xt = jnp.transpose(x, (0, 2, 1))
    dense_args =(wqd, b_pos[None, :], w0t, b0[None, :],
                  W1.T, b1[None, :], W2.T, b2[None, :],
                  codebook, cbt, csq, W3.T, b3[None, :],
                  W4.T, b4[None, :], W_out.T, b_out[None, :])

    outs, commits, counts = [], [], []
    for c in range(_NC):
        qc = lax.slice_in_dim(q, c * _QC, (c + 1) * _QC, axis=1)
        idx = _knn(qc, xt)
        nrow = _B * _QC
        idxt = jnp.transpose(idx.reshape(nrow, _K), (1, 0)).reshape(1, _K * nrow)
        g = _sc_gather(table, idxt)
        g3 = g.reshape(_K, nrow, _OUT + _POS)
        out_c, commit_c, counts_c = _dense(g3, qc.reshape(nrow, 3), *dense_args)
        outs.append(out_c.reshape(_B, _QC, 3))
        commits.append(commit_c)
        counts.append(counts_c)

    out = jnp.concatenate(outs, axis=1)
    commit = sum(jnp.sum(cv) for cv in commits) / (_B * _Q * _HID) * 0.001
    orth = jnp.sum(orth_v) / (_H * _CB * _CB) - (1.0 / _CB)
    loss_vq = commit + 1e-5 * orth
    usage = jnp.sum(sum(counts) > 0.5) / _CB
    return out, loss_vq, usage


# speedup vs baseline: 1.0959x; 1.0959x over previous
import jax
import jax.numpy as jnp
from jax import lax
from jax.experimental import pallas as pl
from jax.experimental.pallas import tpu as pltpu
from jax.experimental.pallas import tpu_sc as plsc

_B, _N, _Q = 2, 4096, 4096
_HID, _K, _POS, _OUT, _CB, _H, _CD = 256, 8, 128, 128, 128, 4, 64
_BN = _B * _N
_BQ = _B * _Q
_TN = 1024
_TK = 512
_TD = 1024
_GW = 128
_NC = 2
_QC = _Q // _NC


def _enc_body(xf, we1, be1, we2, be2, wxp, cb, cbt, table, orth_v):
    step = pl.program_id(0)

    @pl.when(step == 0)
    def _orth():
        o_acc = jnp.zeros((1, _CB), jnp.float32)
        for h in range(_H):
            ch = cb[h]
            cht = cbt[h]
            nrm = jnp.sqrt(jnp.sum(ch * ch, axis=1, keepdims=True))
            nrm_t = jnp.sqrt(jnp.sum(cht * cht, axis=0, keepdims=True))
            nc = ch / (nrm + 1e-8)
            nct = cht / (nrm_t + 1e-8)
            cosm = jnp.dot(nc, nct, preferred_element_type=jnp.float32)
            o_acc = o_acc + jnp.sum(cosm * cosm, axis=0, keepdims=True)
        orth_v[...] = o_acc

    h = jnp.maximum(jnp.dot(xf[...], we1[...], preferred_element_type=jnp.float32)
                    + be1[...], 0.0)
    pts = jnp.dot(h, we2[...], preferred_element_type=jnp.float32) + be2[...]
    p = jnp.dot(xf[...], wxp[...], preferred_element_type=jnp.float32)
    table[:, 0:_OUT] = pts
    table[:, _OUT:_OUT + _POS] = p


def _encoder(xf, we1, be1, we2, be2, wxp, cb, cbt):
    grid = (_BN // _TN,)
    return pl.pallas_call(
        _enc_body,
        grid=grid,
        in_specs=[
            pl.BlockSpec((_TN, 3), lambda i: (i, 0)),
            pl.BlockSpec((3, _OUT), lambda i: (0, 0)),
            pl.BlockSpec((1, _OUT), lambda i: (0, 0)),
            pl.BlockSpec((_OUT, _OUT), lambda i: (0, 0)),
            pl.BlockSpec((1, _OUT), lambda i: (0, 0)),
            pl.BlockSpec((3, _POS), lambda i: (0, 0)),
            pl.BlockSpec((_H, _CB, _CD), lambda i: (0, 0, 0)),
            pl.BlockSpec((_H, _CD, _CB), lambda i: (0, 0, 0)),
        ],
        out_specs=[
            pl.BlockSpec((_TN, _OUT + _POS), lambda i: (i, 0)),
            pl.BlockSpec((1, _CB), lambda i: (0, 0)),
        ],
        out_shape=[
            jax.ShapeDtypeStruct((_BN, _OUT + _POS), jnp.float32),
            jax.ShapeDtypeStruct((1, _CB), jnp.float32),
        ],
    )(xf, we1, be1, we2, be2, wxp, cb, cbt)


_NG = 32
_GL = _N // _NG
_T = 3


def _knn_body(q3, xt, idx):
    b = pl.program_id(0)
    qt = q3[0]
    x = xt[0]
    qx = jnp.dot(qt, x, preferred_element_type=jnp.float32)
    qsq = jnp.sum(qt * qt, axis=1, keepdims=True)
    xsq = jnp.sum(x * x, axis=0, keepdims=True)
    d = qsq - 2.0 * qx + xsq
    cols = [d[:, c * _GL:(c + 1) * _GL] for c in range(_NG)]
    liota = lax.broadcasted_iota(jnp.int32, (_TK, _GL), 1)
    cvs, cis = [], []
    for t in range(_T):
        m = cols[0]
        for c in range(1, _NG):
            m = jnp.minimum(m, cols[c])
        ci = jnp.full((_TK, _GL), _NG, jnp.int32)
        for c in range(_NG):
            eq = cols[c] == m
            ci = jnp.minimum(ci, jnp.where(eq, c, _NG))
            cols[c] = jnp.where(eq, jnp.inf, cols[c])
        cvs.append(m)
        cis.append(ci * _GL + liota)
    off = b * _N
    for k in range(_K):
        m2 = cvs[0]
        for t in range(1, _T):
            m2 = jnp.minimum(m2, cvs[t])
        m = jnp.min(m2, axis=1, keepdims=True)
        ik = jnp.full((_TK, 1), _N, jnp.int32)
        for t in range(_T):
            eq = cvs[t] == m
            cand = jnp.min(jnp.where(eq, cis[t], _N), axis=1, keepdims=True)
            ik = jnp.minimum(ik, cand)
            cvs[t] = jnp.where(eq, jnp.inf, cvs[t])
        idx[0, :, k:k + 1] = ik + off


def _knn(q, xt):
    nq = q.shape[1]
    grid = (_B, nq // _TK)
    return pl.pallas_call(
        _knn_body,
        grid=grid,
        in_specs=[
            pl.BlockSpec((1, _TK, 3), lambda b, i: (b, i, 0)),
            pl.BlockSpec((1, 3, _N), lambda b, i: (b, 0, 0)),
        ],
        out_specs=pl.BlockSpec((1, _TK, _K), lambda b, i: (b, i, 0)),
        out_shape=jax.ShapeDtypeStruct((_B, nq, _K), jnp.int32),
    )(q, xt)


def _sc_gather(table, idxt):
    nidx = idxt.shape[1]
    mesh = plsc.VectorSubcoreMesh(core_axis_name="core",
                                  subcore_axis_name="subcore")

    @pl.kernel(out_type=jax.ShapeDtypeStruct((nidx, _OUT + _POS), jnp.float32),
               mesh=mesh)
    def gk(tab_hbm, i_hbm, o_hbm):
        def body(i_vmem, o_vmem):
            pltpu.sync_copy(tab_hbm.at[i_vmem.at[0]], o_vmem)

        pltpu.emit_pipeline(
            body,
            grid=(nidx // _GW,),
            in_specs=[pl.BlockSpec((1, _GW), index_map=lambda i: (0, i))],
            out_specs=[pl.BlockSpec((_GW, _OUT + _POS),
                                    index_map=lambda i: (i, 0))],
            core_axis_name=("core", "subcore"),
            dimension_semantics=(pltpu.PARALLEL,),
        )(i_hbm, o_hbm)

    return gk(table, idxt)


def _dense_body(g, qf, wqd, bpos, w0t, b0, w1t, b1, w2t, b2,
                cb, cbt, csq, w3t, b3, w4t, b4, wot, bo,
                out, commit_v, counts_v):
    step = pl.program_id(0)

    @pl.when(step == 0)
    def _init():
        commit_v[...] = jnp.zeros_like(commit_v)
        counts_v[...] = jnp.zeros_like(counts_v)

    qt = qf[...]
    qpart = jnp.dot(qt, wqd[...], preferred_element_type=jnp.float32) + bpos[...]
    acc = jnp.zeros((_TD, _HID * 2), jnp.float32)
    for k in range(_K):
        gk = g[k]
        posk = jnp.maximum(qpart + gk[:, _OUT:_OUT + _POS],
                           0.0).astype(jnp.bfloat16)
        acc = acc + jnp.dot(posk, w0t[k, 0:_POS, :],
                            preferred_element_type=jnp.float32)
        acc = acc + jnp.dot(gk[:, 0:_OUT].astype(jnp.bfloat16),
                            w0t[k, _POS:_POS + _OUT, :],
                            preferred_element_type=jnp.float32)
    h0 = jnp.maximum(acc + b0[...], 0.0)
    h1 = jnp.maximum(jnp.dot(h0, w1t[...], preferred_element_type=jnp.float32)
                     + b1[...], 0.0)
    f = jnp.maximum(jnp.dot(h1, w2t[...], preferred_element_type=jnp.float32)
                    + b2[...], 0.0)

    iota = lax.broadcasted_iota(jnp.int32, (_TD, _CB), 1)
    quants = []
    for h in range(_H):
        xh = f[:, h * _CD:(h + 1) * _CD]
        xsq = jnp.sum(xh * xh, axis=1, keepdims=True)
        dh = (xsq - 2.0 * jnp.dot(xh, cbt[h], preferred_element_type=jnp.float32)
              + csq[h])
        m = jnp.min(dh, axis=1, keepdims=True)
        sel = jnp.where(dh == m, iota, _CB)
        ind = jnp.min(sel, axis=1, keepdims=True)
        onehot = (iota == ind).astype(jnp.float32)
        quants.append(jnp.dot(onehot, cb[h], preferred_element_type=jnp.float32))
        if h == 0:
            counts_v[...] += jnp.sum(onehot, axis=0, keepdims=True)
    quant = jnp.concatenate(quants, axis=1)
    err2 = (quant - f) ** 2
    commit_v[...] += jnp.sum(err2, axis=0, keepdims=True)

    net = jnp.maximum(jnp.dot(f, w3t[0:_HID, :], preferred_element_type=jnp.float32)
                      + jnp.dot(quant, w3t[_HID:2 * _HID, :],
                                preferred_element_type=jnp.float32)
                      + b3[...], 0.0)
    net = jnp.maximum(jnp.dot(net, w4t[...], preferred_element_type=jnp.float32)
                      + b4[...], 0.0)
    out[...] = jnp.dot(net, wot[...], preferred_element_type=jnp.float32) + bo[...]


def _dense(g3, qf, wqd, bpos, w0t, b0, w1t, b1, w2t, b2,
           cb, cbt, csq, w3t, b3, w4t, b4, wot, bo):
    nr = qf.shape[0]
    grid = (nr // _TD,)
    const = lambda *shape: pl.BlockSpec(shape, lambda i: tuple(0 for _ in shape))
    return pl.pallas_call(
        _dense_body,
        grid=grid,
        in_specs=[
            pl.BlockSpec((_K, _TD, _OUT + _POS), lambda i: (0, i, 0)),
            pl.BlockSpec((_TD, 3), lambda i: (i, 0)),
            const(3, _POS),
            const(1, _POS),
            const(_K, _OUT + _POS, _HID * 2),
            const(1, _HID * 2),
            const(_HID * 2, _HID),
            const(1, _HID),
            const(_HID, _HID),
            const(1, _HID),
            const(_H, _CB, _CD),
            const(_H, _CD, _CB),
            const(_H, 1, _CB),
            const(_HID * 2, _HID),
            const(1, _HID),
            const(_HID, _HID),
            const(1, _HID),
            const(_HID, 3),
            const(1, 3),
        ],
        out_specs=[
            pl.BlockSpec((_TD, 3), lambda i: (i, 0)),
            pl.BlockSpec((1, _HID), lambda i: (0, 0)),
            pl.BlockSpec((1, _CB), lambda i: (0, 0)),
        ],
        out_shape=[
            jax.ShapeDtypeStruct((nr, 3), jnp.float32),
            jax.ShapeDtypeStruct((1, _HID), jnp.float32),
            jax.ShapeDtypeStruct((1, _CB), jnp.float32),
        ],
    )(g3, qf, wqd, bpos, w0t, b0, w1t, b1, w2t, b2,
      cb, cbt, csq, w3t, b3, w4t, b4, wot, bo)


def kernel(q, x, W_enc1, b_enc1, W_enc2, b_enc2, W_pos, b_pos, W0, b0,
           W1, b1, W2, b2, W3, b3, W4, b4, W_out, b_out, codebook):
    f32 = jnp.float32
    bf16 = jnp.bfloat16
    wxp = W_pos[3:6] + W_pos[6:9]
    wqd = W_pos[0:3] - W_pos[6:9]
    w0t = W0.T.reshape(_K, _OUT + _POS, _HID * 2).astype(bf16)
    cbt = jnp.transpose(codebook, (0, 2, 1))
    csq = jnp.sum(codebook * codebook, axis=-1)[:, None, :]

    xf = x.reshape(_BN, 3)
    table, orth_v = _encoder(xf, W_enc1, b_enc1[None, :], W_enc2, b_enc2[None, :],
                             wxp, codebook, cbt)

    xt = jnp.transpose(x, (0, 2, 1))
    dense_args =(wqd, b_pos[None, :], w0t, b0[None, :],
                  W1.T, b1[None, :], W2.T, b2[None, :],
                  codebook, cbt, csq, W3.T, b3[None, :],
                  W4.T, b4[None, :], W_out.T, b_out[None, :])

    outs, commits, counts = [], [], []
    for c in range(_NC):
        qc = lax.slice_in_dim(q, c * _QC, (c + 1) * _QC, axis=1)
        idx = _knn(qc, xt)
        nrow = _B * _QC
        idxt = jnp.transpose(idx.reshape(nrow, _K), (1, 0)).reshape(1, _K * nrow)
        g = _sc_gather(table, idxt)
        g3 = g.reshape(_K, nrow, _OUT + _POS)
        out_c, commit_c, counts_c = _dense(g3, qc.reshape(nrow, 3), *dense_args)
        outs.append(out_c.reshape(_B, _QC, 3))
        commits.append(commit_c)
        counts.append(counts_c)

    out = jnp.concatenate(outs, axis=1)
    commit = sum(jnp.sum(cv) for cv in commits) / (_B * _Q * _HID) * 0.001
    orth = jnp.sum(orth_v) / (_H * _CB * _CB) - (1.0 / _CB)
    loss_vq = commit + 1e-5 * orth
    usage = jnp.sum(sum(counts) > 0.5) / _CB
    return out, loss_vq, usage
